# baseline (speedup 1.0000x reference)
import functools

import jax
import jax.numpy as jnp
from jax import lax
from jax.experimental import pallas as pl
from jax.experimental.pallas import tpu as pltpu
from jax.experimental.pallas import tpu_sc as plsc

VOCAB = 1_000_000
EMBED = 64
ROWS = 4096
COLS = 200
B = ROWS * COLS

NUM_CORES = 2
NUM_SUBCORES = 16
NW = NUM_CORES * NUM_SUBCORES
BPW = B // NW

G = 128
K = 8
C = G * K
NCHUNK = BPW // C

_mesh = plsc.VectorSubcoreMesh(core_axis_name="c", subcore_axis_name="s")


@functools.partial(
    pl.kernel,
    mesh=_mesh,
    compiler_params=pltpu.CompilerParams(use_tc_tiling_on_sc=False),
    out_type=jax.ShapeDtypeStruct((B // G, G, EMBED), jnp.float32),
    scratch_types=[
        pltpu.VMEM((K, G), jnp.int32),
        pltpu.VMEM((K, G, EMBED), jnp.float32),
        pltpu.SemaphoreType.DMA,
    ],
)
def _emb_lookup(idx_hbm, w_hbm, out_hbm, idx_v, rows_v, sem):
    wid = lax.axis_index("s") * NUM_CORES + lax.axis_index("c")
    base = wid * (BPW // G)

    def body(g, carry):
        r0 = base + g * K
        pltpu.sync_copy(idx_hbm.at[pl.ds(r0, K)], idx_v)
        copies = []
        for j in range(K):
            copies.append(
                pltpu.async_copy(w_hbm.at[idx_v.at[j]], rows_v.at[j], sem)
            )
        for cp in copies:
            cp.wait()
        pltpu.sync_copy(rows_v, out_hbm.at[pl.ds(r0, K)])
        return carry

    lax.fori_loop(0, NCHUNK, body, 0)


def kernel(indices, weight):
    idx = indices.reshape(B // G, G)
    out = _emb_lookup(idx, weight)
    return out.reshape(ROWS, COLS, EMBED)
